# baseline (speedup 1.0000x reference)
import functools
import math

import jax
import jax.numpy as jnp
from jax import lax
from jax.experimental import pallas as pl
from jax.experimental.pallas import tpu as pltpu
from jax.experimental.pallas import tpu_sc as plsc

_NC, _NS = 2, 16
_CH = 128
_L = 16
_RU = 4

_LOG2PI = math.log(2.0 * math.pi)
_C1 = math.log(1.0 / 5.0 - 1.0 / 20.0) - 0.5 * _LOG2PI
_C2 = math.log(4.0 / 5.0 - 1.0 / 20.0) - 0.5 * _LOG2PI
_C3 = math.log(1.0 / 10.0) - 0.5 * _LOG2PI
_LN2 = math.log(2.0)
_P = (-0.02482560661573763, 0.2668588228733046, -1.2342631730840539,
      3.218832837151793, -5.264110477180775, 6.06583014324084,
      -3.0283174810522704)


def _sc_mixture(user_mu, user_logvar, z, idx2, mu_p, lv_p, lv_u):
    V, D = user_mu.shape
    B = z.shape[0]
    nw = _NC * _NS
    n_ch = idx2.shape[0] // nw
    b_per_w = n_ch * _CH
    nc8 = D // _L
    mesh = plsc.VectorSubcoreMesh(core_axis_name="c", subcore_axis_name="s")

    @functools.partial(
        pl.kernel,
        mesh=mesh,
        out_type=jax.ShapeDtypeStruct((B, D), jnp.float32),
        scratch_types=[
            pltpu.VMEM((n_ch, _CH), jnp.int32),
            pltpu.VMEM((2, _CH, D), jnp.float32),
            pltpu.VMEM((2, _CH, D), jnp.float32),
            pltpu.VMEM((2, _CH, D), jnp.float32),
            pltpu.VMEM((3, D), jnp.float32),
            pltpu.VMEM((4, D), jnp.float32),
            pltpu.SemaphoreType.DMA,
            pltpu.SemaphoreType.DMA,
            pltpu.SemaphoreType.DMA,
            pltpu.SemaphoreType.DMA,
        ],
    )
    def k(mu_hbm, lv_hbm, z_hbm, idx_hbm, mup_hbm, lvp_hbm, lvu_hbm, out_hbm,
          idx_v, mu_v, lv_v, z_v, pri_v, cst_v,
          in_sem0, in_sem1, out_sem0, out_sem1):
        wid = lax.axis_index("s") * _NC + lax.axis_index("c")
        base = wid * b_per_w
        in_sems = (in_sem0, in_sem1)
        out_sems = (out_sem0, out_sem1)
        pltpu.sync_copy(idx_hbm.at[pl.ds(wid * n_ch, n_ch)], idx_v)
        pltpu.sync_copy(mup_hbm, pri_v.at[0])
        pltpu.sync_copy(lvp_hbm, pri_v.at[1])
        pltpu.sync_copy(lvu_hbm, pri_v.at[2])
        for c in range(nc8):
            cs = pl.ds(c * _L, _L)
            lvp = pri_v[1, cs]
            lvu = pri_v[2, cs]
            cst_v[0, cs] = -0.5 * jnp.exp(-lvp)
            cst_v[1, cs] = _C1 - 0.5 * lvp
            cst_v[2, cs] = -0.5 * jnp.exp(-lvu)
            cst_v[3, cs] = _C3 - 0.5 * lvu

        def fire(ch):
            p = ch % 2
            rows = pl.ds(base + ch * _CH, _CH)
            return (
                pltpu.async_copy(mu_hbm.at[idx_v.at[ch]], mu_v.at[p], in_sems[p]),
                pltpu.async_copy(lv_hbm.at[idx_v.at[ch]], lv_v.at[p], in_sems[p]),
                pltpu.async_copy(z_hbm.at[rows], z_v.at[p], in_sems[p]),
            )

        def compute(ch):
            p = ch % 2

            @plsc.parallel_loop(0, _CH, step=1, unroll=_RU)
            def _(r):
                for c in range(nc8):
                    cs = pl.ds(c * _L, _L)
                    zv = z_v[p, r, cs]
                    mu = mu_v[p, r, cs]
                    lv = lv_v[p, r, cs]
                    t = zv - mu
                    e2 = jnp.exp(_C2 - 0.5 * (lv + t * t * jnp.exp(-lv)))
                    zp = zv - pri_v[0, cs]
                    zp2 = zp * zp
                    e1 = jnp.exp(cst_v[0, cs] * zp2 + cst_v[1, cs])
                    e3 = jnp.exp(cst_v[2, cs] * zp2 + cst_v[3, cs])
                    s = e1 + e2 + e3
                    bits = lax.bitcast_convert_type(s, jnp.int32)
                    ei = lax.shift_right_arithmetic(bits, 23) - 127
                    mant = lax.bitcast_convert_type(
                        (bits & 0x007FFFFF) | 0x3F800000, jnp.float32)
                    pm = jnp.full((_L,), _P[0], jnp.float32)
                    for coef in _P[1:]:
                        pm = pm * mant + coef
                    z_v[p, r, cs] = (ei.astype(jnp.float32) + pm) * _LN2

        in_cps = {0: fire(0)}
        out_cps = {}
        for ch in range(n_ch):
            p = ch % 2
            for cp in in_cps.pop(ch):
                cp.wait()
            if ch >= 1:
                out_cps.pop(ch - 1).wait()
            if ch + 1 < n_ch:
                in_cps[ch + 1] = fire(ch + 1)
            compute(ch)
            rows = pl.ds(base + ch * _CH, _CH)
            out_cps[ch] = pltpu.async_copy(z_v.at[p], out_hbm.at[rows], out_sems[p])
        out_cps.pop(n_ch - 1).wait()

    return k(user_mu, user_logvar, z, idx2, mu_p, lv_p, lv_u)


def kernel(z, idx, mu_prior, logvar_prior, logvar_uniform_prior, user_mu, user_logvar):
    B, D = z.shape
    idx2 = idx.astype(jnp.int32).reshape(-1, _CH)
    return _sc_mixture(
        user_mu,
        user_logvar,
        z,
        idx2,
        mu_prior.reshape(D),
        logvar_prior.reshape(D),
        logvar_uniform_prior.reshape(D),
    )


# speedup vs baseline: 3.2496x; 3.2496x over previous
import functools
import math

import jax
import jax.numpy as jnp
from jax import lax
from jax.experimental import pallas as pl
from jax.experimental.pallas import tpu as pltpu
from jax.experimental.pallas import tpu_sc as plsc

_NC, _NS = 2, 16
_CH = 128
_L = 16
_HALVES = 2

_LOG2PI = math.log(2.0 * math.pi)
_C1 = math.log(1.0 / 5.0 - 1.0 / 20.0) - 0.5 * _LOG2PI
_C2 = math.log(4.0 / 5.0 - 1.0 / 20.0) - 0.5 * _LOG2PI
_C3 = math.log(1.0 / 10.0) - 0.5 * _LOG2PI


def _sc_gather_e2(user_mu, user_logvar, idx2, z):
    V, D = user_mu.shape
    B = z.shape[0]
    nw = _NC * _NS
    n_ch = idx2.shape[0] // nw
    b_per_w = n_ch * _CH
    mesh = plsc.VectorSubcoreMesh(core_axis_name="c", subcore_axis_name="s")

    @functools.partial(
        pl.kernel,
        mesh=mesh,
        out_type=jax.ShapeDtypeStruct((B, D), jnp.float32),
        scratch_types=[
            pltpu.VMEM((n_ch, _CH), jnp.int32),
            pltpu.VMEM((2, _CH, D), jnp.float32),
            pltpu.VMEM((2, _CH, D), jnp.float32),
            pltpu.VMEM((2, _CH, D), jnp.float32),
            pltpu.SemaphoreType.DMA,
            pltpu.SemaphoreType.DMA,
            pltpu.SemaphoreType.DMA,
            pltpu.SemaphoreType.DMA,
        ],
    )
    def k(mu_hbm, lv_hbm, z_hbm, idx_hbm, e2_out, idx_v, mu_v, lv_v, z_v,
          in_sem0, in_sem1, out_sem0, out_sem1):
        wid = lax.axis_index("s") * _NC + lax.axis_index("c")
        base = wid * b_per_w
        in_sems = (in_sem0, in_sem1)
        out_sems = (out_sem0, out_sem1)
        pltpu.sync_copy(idx_hbm.at[pl.ds(wid * n_ch, n_ch)], idx_v)

        def fire(ch):
            p = ch % 2
            rows = pl.ds(base + ch * _CH, _CH)
            return (
                pltpu.async_copy(mu_hbm.at[idx_v.at[ch]], mu_v.at[p], in_sems[p]),
                pltpu.async_copy(lv_hbm.at[idx_v.at[ch]], lv_v.at[p], in_sems[p]),
                pltpu.async_copy(z_hbm.at[rows], z_v.at[p], in_sems[p]),
            )

        def compute(ch):
            p = ch % 2

            @plsc.parallel_loop(0, _CH, step=1, unroll=4)
            def _(r):
                for c in range(D // _L):
                    cs = pl.ds(c * _L, _L)
                    zv = z_v[p, r, cs]
                    mu = mu_v[p, r, cs]
                    lv = lv_v[p, r, cs]
                    t = zv - mu
                    acc = lv + t * t * jnp.exp(-lv)
                    z_v[p, r, cs] = jnp.exp(_C2 - 0.5 * acc)

        in_cps = {0: fire(0)}
        out_cps = {}
        for ch in range(n_ch):
            p = ch % 2
            for cp in in_cps.pop(ch):
                cp.wait()
            if ch >= 1:
                out_cps.pop(ch - 1).wait()
            if ch + 1 < n_ch:
                in_cps[ch + 1] = fire(ch + 1)
            compute(ch)
            rows = pl.ds(base + ch * _CH, _CH)
            out_cps[ch] = pltpu.async_copy(z_v.at[p], e2_out.at[rows], out_sems[p])
        out_cps.pop(n_ch - 1).wait()

    return k(user_mu, user_logvar, z, idx2)


def _tc_math(z_h, e2_h, mu_p, lv_p, lv_u, half, n_halves, B, prev=None):
    Bh, D = z_h.shape
    blk = 2048
    nblk = Bh // blk

    def body(z_ref, e2_ref, mup_ref, lvp_ref, lvu_ref, *rest):
        o_ref = rest[-1]
        mup = mup_ref[...]
        lvp = lvp_ref[...]
        lvu = lvu_ref[...]
        a1 = -0.5 * jnp.exp(-lvp)
        b1 = _C1 - 0.5 * lvp
        a3 = -0.5 * jnp.exp(-lvu)
        b3 = _C3 - 0.5 * lvu
        zp2 = (z_ref[...] - mup) ** 2
        e1 = jnp.exp(a1 * zp2 + b1)
        e3 = jnp.exp(a3 * zp2 + b3)
        o_ref[...] = jnp.log(e1 + e2_ref[...] + e3)

    bs = pl.BlockSpec((blk, D), lambda i: (i, 0))
    ps = pl.BlockSpec((1, D), lambda i: (0, 0))
    out_spec = pl.BlockSpec((blk, D), lambda i, _h=half: (i + _h * nblk, 0))
    in_specs = [bs, bs, ps, ps, ps]
    args = [z_h, e2_h, mu_p, lv_p, lv_u]
    aliases = {}
    if prev is not None:
        in_specs.append(pl.BlockSpec(memory_space=pl.ANY))
        args.append(prev)
        aliases = {5: 0}
    return pl.pallas_call(
        body,
        grid=(nblk,),
        in_specs=in_specs,
        out_specs=out_spec,
        out_shape=jax.ShapeDtypeStruct((B, D), jnp.float32),
        input_output_aliases=aliases,
    )(*args)


def kernel(z, idx, mu_prior, logvar_prior, logvar_uniform_prior, user_mu, user_logvar):
    B, D = z.shape
    idx32 = idx.astype(jnp.int32)
    mu_p = mu_prior.reshape(1, D)
    lv_p = logvar_prior.reshape(1, D)
    lv_u = logvar_uniform_prior.reshape(1, D)
    Bh = B // _HALVES
    e2s = []
    for h in range(_HALVES):
        sl = slice(h * Bh, (h + 1) * Bh)
        idx2 = idx32[sl].reshape(-1, _CH)
        e2s.append(_sc_gather_e2(user_mu, user_logvar, idx2, z[sl]))
    out = None
    for h in range(_HALVES):
        sl = slice(h * Bh, (h + 1) * Bh)
        out = _tc_math(z[sl], e2s[h], mu_p, lv_p, lv_u, h, _HALVES, B, prev=out)
    return out


# speedup vs baseline: 3.6555x; 1.1249x over previous
import functools
import math

import jax
import jax.numpy as jnp
from jax import lax
from jax.experimental import pallas as pl
from jax.experimental.pallas import tpu as pltpu
from jax.experimental.pallas import tpu_sc as plsc

_NC, _NS = 2, 16
_CH = 128
_L = 16

_LOG2PI = math.log(2.0 * math.pi)
_C1 = math.log(1.0 / 5.0 - 1.0 / 20.0) - 0.5 * _LOG2PI
_C2 = math.log(4.0 / 5.0 - 1.0 / 20.0) - 0.5 * _LOG2PI
_C3 = math.log(1.0 / 10.0) - 0.5 * _LOG2PI


def _sc_gather_e2(user_mu, user_logvar, idx2, z):
    V, D = user_mu.shape
    B = z.shape[0]
    nw = _NC * _NS
    n_ch = idx2.shape[0] // nw
    b_per_w = n_ch * _CH
    mesh = plsc.VectorSubcoreMesh(core_axis_name="c", subcore_axis_name="s")

    @functools.partial(
        pl.kernel,
        mesh=mesh,
        out_type=jax.ShapeDtypeStruct((B, D), jnp.float32),
        scratch_types=[
            pltpu.VMEM((n_ch, _CH), jnp.int32),
            pltpu.VMEM((2, _CH, D), jnp.float32),
            pltpu.VMEM((2, _CH, D), jnp.float32),
            pltpu.VMEM((2, _CH, D), jnp.float32),
            pltpu.SemaphoreType.DMA,
            pltpu.SemaphoreType.DMA,
            pltpu.SemaphoreType.DMA,
            pltpu.SemaphoreType.DMA,
        ],
    )
    def k(mu_hbm, lv_hbm, z_hbm, idx_hbm, e2_out, idx_v, mu_v, lv_v, z_v,
          in_sem0, in_sem1, out_sem0, out_sem1):
        wid = lax.axis_index("s") * _NC + lax.axis_index("c")
        base = wid * b_per_w
        in_sems = (in_sem0, in_sem1)
        out_sems = (out_sem0, out_sem1)
        pltpu.sync_copy(idx_hbm.at[pl.ds(wid * n_ch, n_ch)], idx_v)

        def fire(ch):
            p = ch % 2
            rows = pl.ds(base + ch * _CH, _CH)
            return (
                pltpu.async_copy(mu_hbm.at[idx_v.at[ch]], mu_v.at[p], in_sems[p]),
                pltpu.async_copy(lv_hbm.at[idx_v.at[ch]], lv_v.at[p], in_sems[p]),
                pltpu.async_copy(z_hbm.at[rows], z_v.at[p], in_sems[p]),
            )

        def compute(ch):
            p = ch % 2

            @plsc.parallel_loop(0, _CH, step=1, unroll=4)
            def _(r):
                for c in range(D // _L):
                    cs = pl.ds(c * _L, _L)
                    zv = z_v[p, r, cs]
                    mu = mu_v[p, r, cs]
                    lv = lv_v[p, r, cs]
                    t = zv - mu
                    acc = lv + t * t * jnp.exp(-lv)
                    z_v[p, r, cs] = jnp.exp(_C2 - 0.5 * acc)

        in_cps = {0: fire(0)}
        out_cps = {}
        for ch in range(n_ch):
            p = ch % 2
            for cp in in_cps.pop(ch):
                cp.wait()
            if ch >= 1:
                out_cps.pop(ch - 1).wait()
            if ch + 1 < n_ch:
                in_cps[ch + 1] = fire(ch + 1)
            compute(ch)
            rows = pl.ds(base + ch * _CH, _CH)
            out_cps[ch] = pltpu.async_copy(z_v.at[p], e2_out.at[rows], out_sems[p])
        out_cps.pop(n_ch - 1).wait()

    return k(user_mu, user_logvar, z, idx2)


def _tc_prior_sum(z, mu_p, lv_p, lv_u):
    B, D = z.shape
    blk = 2048

    def body(z_ref, mup_ref, lvp_ref, lvu_ref, o_ref):
        mup = mup_ref[...]
        lvp = lvp_ref[...]
        lvu = lvu_ref[...]
        a1 = -0.5 * jnp.exp(-lvp)
        b1 = _C1 - 0.5 * lvp
        a3 = -0.5 * jnp.exp(-lvu)
        b3 = _C3 - 0.5 * lvu
        zp2 = (z_ref[...] - mup) ** 2
        o_ref[...] = jnp.exp(a1 * zp2 + b1) + jnp.exp(a3 * zp2 + b3)

    bs = pl.BlockSpec((blk, D), lambda i: (i, 0))
    ps = pl.BlockSpec((1, D), lambda i: (0, 0))
    return pl.pallas_call(
        body,
        grid=(B // blk,),
        in_specs=[bs, ps, ps, ps],
        out_specs=bs,
        out_shape=jax.ShapeDtypeStruct((B, D), jnp.float32),
    )(z, mu_p, lv_p, lv_u)


def _tc_logsum(t, e2):
    B, D = t.shape
    blk = 2048

    def body(t_ref, e2_ref, o_ref):
        o_ref[...] = jnp.log(t_ref[...] + e2_ref[...])

    bs = pl.BlockSpec((blk, D), lambda i: (i, 0))
    return pl.pallas_call(
        body,
        grid=(B // blk,),
        in_specs=[bs, bs],
        out_specs=bs,
        out_shape=jax.ShapeDtypeStruct((B, D), jnp.float32),
    )(t, e2)


def kernel(z, idx, mu_prior, logvar_prior, logvar_uniform_prior, user_mu, user_logvar):
    B, D = z.shape
    idx2 = idx.astype(jnp.int32).reshape(-1, _CH)
    e2 = _sc_gather_e2(user_mu, user_logvar, idx2, z)
    t = _tc_prior_sum(
        z,
        mu_prior.reshape(1, D),
        logvar_prior.reshape(1, D),
        logvar_uniform_prior.reshape(1, D),
    )
    return _tc_logsum(t, e2)
